# baseline (speedup 1.0000x reference)
import functools

import jax
import jax.numpy as jnp
from jax import lax
from jax.experimental import pallas as pl
from jax.experimental.pallas import tpu as pltpu
from jax.experimental.pallas import tpu_sc as plsc

_PMT = 16
_D = 1024
_S = 688
_SB = _S // _PMT
_NW = 32
_RPW = 2


def _gather_body(table_hbm, idx_hbm, out_hbm, idx_v, rows_v, sem):
    wid = lax.axis_index("s") * 2 + lax.axis_index("c")
    pltpu.sync_copy(idx_hbm.at[wid], idx_v)
    pltpu.async_copy(table_hbm.at[idx_v], rows_v, sem).wait()
    pltpu.sync_copy(rows_v, out_hbm.at[pl.ds(wid * _RPW, _RPW)])


def _sc_gather(table, idx):
    mesh = plsc.VectorSubcoreMesh(core_axis_name="c", subcore_axis_name="s")
    k = functools.partial(
        pl.kernel,
        mesh=mesh,
        out_type=jax.ShapeDtypeStruct((_NW * _RPW, _D), jnp.float32),
        scratch_types=[
            pltpu.VMEM((_RPW,), jnp.int32),
            pltpu.VMEM((_RPW, _D), jnp.float32),
            pltpu.SemaphoreType.DMA,
        ],
    )(_gather_body)
    return k(table, idx)


def _concat_body(p_ref, tail_ref, head_ref, mask_ref, xo_ref, mo_ref):
    i = pl.program_id(1)

    @pl.when(i == 0)
    def _():
        xo_ref[:, :_PMT, :] = p_ref[...]

    @pl.when(i > 0)
    def _():
        xo_ref[:, :_PMT, :] = tail_ref[...]

    xo_ref[:, _PMT:, :] = head_ref[:, : _S - _PMT, :]
    mo_ref[:, :, :_PMT] = jnp.zeros((1, 1, _PMT), mask_ref.dtype)
    mo_ref[:, :, _PMT:] = mask_ref[...]


def _tc_concat(p3, src_embs, src_mask):
    bs, seq, d = src_embs.shape
    out_seq = seq + _PMT
    return pl.pallas_call(
        _concat_body,
        grid=(bs, out_seq // _S),
        in_specs=[
            pl.BlockSpec((1, _PMT, d), lambda b, i: (b, 0, 0)),
            pl.BlockSpec((1, _PMT, d),
                         lambda b, i: (b, jnp.maximum(i * _SB - 1, 0), 0)),
            pl.BlockSpec((1, _S, d), lambda b, i: (b, i, 0)),
            pl.BlockSpec((1, 1, seq), lambda b, i: (b, 0, 0)),
        ],
        out_specs=[
            pl.BlockSpec((1, _S, d), lambda b, i: (b, i, 0)),
            pl.BlockSpec((1, 1, out_seq), lambda b, i: (b, 0, 0)),
        ],
        out_shape=[
            jax.ShapeDtypeStruct((bs, out_seq, d), src_embs.dtype),
            jax.ShapeDtypeStruct((bs, 1, out_seq), src_mask.dtype),
        ],
    )(p3, src_embs, src_embs, src_mask.reshape(bs, 1, seq))


def kernel(src_embs, src_mask, tissue_idx, prompt_emb):
    bs, seq, d = src_embs.shape
    table = prompt_emb.reshape(-1, d)
    idx = (tissue_idx[:, None] * _PMT
           + jnp.arange(_PMT, dtype=jnp.int32)).reshape(_NW, _RPW)
    p = _sc_gather(table, idx)
    x, new_mask = _tc_concat(p.reshape(bs, _PMT, d), src_embs, src_mask)
    return (x, new_mask.reshape(bs, seq + _PMT))


# speedup vs baseline: 2.4256x; 2.4256x over previous
import functools

import jax
import jax.numpy as jnp
from jax import lax
from jax.experimental import pallas as pl
from jax.experimental.pallas import tpu as pltpu
from jax.experimental.pallas import tpu_sc as plsc

_PMT = 16
_D = 1024
_S = 688
_SB = _S // _PMT
_NW = 32
_RPW = 2


def _gather_body(table_hbm, idx_hbm, out_hbm, idx_v, row_v, sem):
    wid = lax.axis_index("s") * 2 + lax.axis_index("c")
    nb = out_hbm.shape[0]

    @pl.when(wid < nb)
    def _():
        pltpu.sync_copy(idx_hbm.at[wid], idx_v)
        pltpu.async_copy(table_hbm.at[idx_v], row_v, sem).wait()
        pltpu.sync_copy(row_v, out_hbm.at[pl.ds(wid, 1)])


def _sc_gather(table, idx):
    bs = idx.shape[0]
    mesh = plsc.VectorSubcoreMesh(core_axis_name="c", subcore_axis_name="s")
    k = functools.partial(
        pl.kernel,
        mesh=mesh,
        out_type=jax.ShapeDtypeStruct((bs, table.shape[1]), jnp.float32),
        scratch_types=[
            pltpu.VMEM((1,), jnp.int32),
            pltpu.VMEM((1, table.shape[1]), jnp.float32),
            pltpu.SemaphoreType.DMA,
        ],
    )(_gather_body)
    return k(table, idx)


def _concat_body(p_ref, tail_ref, head_ref, mask_ref, xo_ref, mo_ref):
    i = pl.program_id(1)

    @pl.when(i == 0)
    def _():
        xo_ref[:, :_PMT, :] = p_ref[...]

    @pl.when(i > 0)
    def _():
        xo_ref[:, :_PMT, :] = tail_ref[...]

    xo_ref[:, _PMT:, :] = head_ref[:, : _S - _PMT, :]
    mo_ref[:, :, :_PMT] = jnp.zeros((1, 1, _PMT), mask_ref.dtype)
    mo_ref[:, :, _PMT:] = mask_ref[...]


def _tc_concat(p3, src_embs, src_mask):
    bs, seq, d = src_embs.shape
    out_seq = seq + _PMT
    return pl.pallas_call(
        _concat_body,
        grid=(bs, out_seq // _S),
        in_specs=[
            pl.BlockSpec((1, _PMT, d), lambda b, i: (b, 0, 0)),
            pl.BlockSpec((1, _PMT, d),
                         lambda b, i: (b, jnp.maximum(i * _SB - 1, 0), 0)),
            pl.BlockSpec((1, _S, d), lambda b, i: (b, i, 0)),
            pl.BlockSpec((1, 1, seq), lambda b, i: (b, 0, 0)),
        ],
        out_specs=[
            pl.BlockSpec((1, _S, d), lambda b, i: (b, i, 0)),
            pl.BlockSpec((1, 1, out_seq), lambda b, i: (b, 0, 0)),
        ],
        out_shape=[
            jax.ShapeDtypeStruct((bs, out_seq, d), src_embs.dtype),
            jax.ShapeDtypeStruct((bs, 1, out_seq), src_mask.dtype),
        ],
    )(p3, src_embs, src_embs, src_mask.reshape(bs, 1, seq))


def kernel(src_embs, src_mask, tissue_idx, prompt_emb):
    bs, seq, d = src_embs.shape
    p = _sc_gather(prompt_emb, tissue_idx.reshape(bs, 1))
    x, new_mask = _tc_concat(p.reshape(bs, _PMT, d), src_embs, src_mask)
    return (x, new_mask.reshape(bs, seq + _PMT))


# speedup vs baseline: 2.5872x; 1.0666x over previous
import functools

import jax
import jax.numpy as jnp
from jax import lax
from jax.experimental import pallas as pl
from jax.experimental.pallas import tpu as pltpu
from jax.experimental.pallas import tpu_sc as plsc

_PMT = 16
_D = 1024
_S = 688
_SB = _S // _PMT
_NW = 32
_RPW = 2


def _gather_body(table_hbm, idx_hbm, out_hbm, idx_v, row_v, sem):
    wid = lax.axis_index("s") * 2 + lax.axis_index("c")
    nb = out_hbm.shape[0]

    @pl.when(wid < nb)
    def _():
        pltpu.sync_copy(idx_hbm.at[wid], idx_v)
        pltpu.async_copy(table_hbm.at[idx_v], row_v, sem).wait()
        pltpu.sync_copy(row_v, out_hbm.at[pl.ds(wid, 1)])


def _sc_gather(table, idx):
    bs = idx.shape[0]
    mesh = plsc.VectorSubcoreMesh(core_axis_name="c", subcore_axis_name="s")
    k = functools.partial(
        pl.kernel,
        mesh=mesh,
        out_type=jax.ShapeDtypeStruct((bs, table.shape[1]), jnp.float32),
        scratch_types=[
            pltpu.VMEM((1,), jnp.int32),
            pltpu.VMEM((1, table.shape[1]), jnp.float32),
            pltpu.SemaphoreType.DMA,
        ],
    )(_gather_body)
    return k(table, idx)


def _concat_body(p_ref, tail_ref, head_ref, mask_ref, xo_ref, mo_ref):
    b = pl.program_id(0)
    i = pl.program_id(1)

    @pl.when(i == 0)
    def _():
        for r in range(_PMT):
            xo_ref[:, r, :] = p_ref[pl.ds(b, 1), pl.ds(r * _D, _D)]

    @pl.when((b == 0) & (i == 0))
    def _():
        nb = mask_ref.shape[0]
        mo_ref[:, :_PMT] = jnp.zeros((nb, _PMT), mask_ref.dtype)
        mo_ref[:, _PMT:] = mask_ref[...]

    @pl.when(i > 0)
    def _():
        xo_ref[:, :_PMT, :] = tail_ref[...]

    xo_ref[:, _PMT:, :] = head_ref[:, : _S - _PMT, :]


def _tc_concat(p, src_embs, src_mask):
    bs, seq, d = src_embs.shape
    out_seq = seq + _PMT
    return pl.pallas_call(
        _concat_body,
        grid=(bs, out_seq // _S),
        in_specs=[
            pl.BlockSpec((bs, _PMT * d), lambda b, i: (0, 0)),
            pl.BlockSpec((1, _PMT, d),
                         lambda b, i: (b, jnp.maximum(i * _SB - 1, 0), 0)),
            pl.BlockSpec((1, _S, d), lambda b, i: (b, i, 0)),
            pl.BlockSpec((bs, seq), lambda b, i: (0, 0)),
        ],
        out_specs=[
            pl.BlockSpec((1, _S, d), lambda b, i: (b, i, 0)),
            pl.BlockSpec((bs, out_seq), lambda b, i: (0, 0)),
        ],
        out_shape=[
            jax.ShapeDtypeStruct((bs, out_seq, d), src_embs.dtype),
            jax.ShapeDtypeStruct((bs, out_seq), src_mask.dtype),
        ],
    )(p, src_embs, src_embs, src_mask)


def kernel(src_embs, src_mask, tissue_idx, prompt_emb):
    bs, seq, d = src_embs.shape
    p = _sc_gather(prompt_emb, tissue_idx.reshape(bs, 1))
    return _tc_concat(p, src_embs, src_mask)


# speedup vs baseline: 2.7616x; 1.0674x over previous
import functools

import jax
import jax.numpy as jnp
from jax import lax
from jax.experimental import pallas as pl
from jax.experimental.pallas import tpu as pltpu
from jax.experimental.pallas import tpu_sc as plsc

_PMT = 16
_D = 1024
_S = 688
_SB = _S // _PMT


def _gather_body(table_hbm, idx_hbm, out_hbm, idx_v, rows_v, sem):
    wid = lax.axis_index("s") * 2 + lax.axis_index("c")
    nb = out_hbm.shape[0]

    @pl.when(wid < nb)
    def _():
        pltpu.sync_copy(idx_hbm, idx_v)
        pltpu.async_copy(table_hbm.at[idx_v], rows_v, sem).wait()
        pltpu.sync_copy(rows_v.at[pl.ds(wid, 1)], out_hbm.at[pl.ds(wid, 1)])


def _sc_gather(table, idx):
    bs = idx.shape[0]
    mesh = plsc.VectorSubcoreMesh(core_axis_name="c", subcore_axis_name="s")
    k = functools.partial(
        pl.kernel,
        mesh=mesh,
        out_type=jax.ShapeDtypeStruct((bs, table.shape[1]), jnp.float32),
        scratch_types=[
            pltpu.VMEM((bs,), jnp.int32),
            pltpu.VMEM((bs, table.shape[1]), jnp.float32),
            pltpu.SemaphoreType.DMA,
        ],
    )(_gather_body)
    return k(table, idx)


def _copy_body(tail_ref, head_ref, mask_ref, xo_ref, mo_ref):
    b = pl.program_id(0)
    i = pl.program_id(1)

    @pl.when(i > 0)
    def _():
        xo_ref[:, :_PMT, :] = tail_ref[...]

    xo_ref[:, _PMT:, :] = head_ref[:, : _S - _PMT, :]

    @pl.when((b == 0) & (i == 0))
    def _():
        nb = mask_ref.shape[0]
        mo_ref[:, :_PMT] = jnp.zeros((nb, _PMT), mask_ref.dtype)
        mo_ref[:, _PMT:] = mask_ref[...]


def _tc_copy(src_embs, src_mask):
    bs, seq, d = src_embs.shape
    out_seq = seq + _PMT
    return pl.pallas_call(
        _copy_body,
        grid=(bs, out_seq // _S),
        in_specs=[
            pl.BlockSpec((1, _PMT, d),
                         lambda b, i: (b, jnp.maximum(i * _SB - 1, 0), 0)),
            pl.BlockSpec((1, _S, d), lambda b, i: (b, i, 0)),
            pl.BlockSpec((bs, seq), lambda b, i: (0, 0)),
        ],
        out_specs=[
            pl.BlockSpec((1, _S, d), lambda b, i: (b, i, 0)),
            pl.BlockSpec((bs, out_seq), lambda b, i: (0, 0)),
        ],
        out_shape=[
            jax.ShapeDtypeStruct((bs, out_seq, d), src_embs.dtype),
            jax.ShapeDtypeStruct((bs, out_seq), src_mask.dtype),
        ],
    )(src_embs, src_embs, src_mask)


def _patch_body(p_ref, xin_ref, xo_ref):
    b = pl.program_id(0)
    for r in range(_PMT):
        xo_ref[:, r, :] = p_ref[pl.ds(b, 1), pl.ds(r * _D, _D)]


def _tc_patch(p, x):
    bs, out_seq, d = x.shape
    return pl.pallas_call(
        _patch_body,
        grid=(bs,),
        in_specs=[
            pl.BlockSpec((bs, _PMT * d), lambda b: (0, 0)),
            pl.BlockSpec(memory_space=pl.ANY),
        ],
        out_specs=pl.BlockSpec((1, _PMT, d), lambda b: (b, 0, 0)),
        out_shape=jax.ShapeDtypeStruct(x.shape, x.dtype),
        input_output_aliases={1: 0},
    )(p, x)


def kernel(src_embs, src_mask, tissue_idx, prompt_emb):
    p = _sc_gather(prompt_emb, tissue_idx)
    x_partial, new_mask = _tc_copy(src_embs, src_mask)
    x = _tc_patch(p, x_partial)
    return (x, new_mask)


# speedup vs baseline: 3.0128x; 1.0910x over previous
import functools

import jax
import jax.numpy as jnp
from jax import lax
from jax.experimental import pallas as pl
from jax.experimental.pallas import tpu as pltpu
from jax.experimental.pallas import tpu_sc as plsc

_PMT = 16
_D = 1024
_S = 688
_SB = _S // _PMT


def _gather_body(table_hbm, idx_hbm, out_hbm, idx_v, rows_v, sem):
    wid = lax.axis_index("s") * 2 + lax.axis_index("c")
    nb = out_hbm.shape[0]

    @pl.when(wid < nb)
    def _():
        pltpu.sync_copy(idx_hbm, idx_v)
        pltpu.async_copy(table_hbm.at[idx_v], rows_v, sem).wait()
        pltpu.sync_copy(rows_v.at[pl.ds(wid, 1)], out_hbm.at[pl.ds(wid, 1)])


def _sc_gather(table, idx):
    bs = idx.shape[0]
    mesh = plsc.VectorSubcoreMesh(core_axis_name="c", subcore_axis_name="s")
    k = functools.partial(
        pl.kernel,
        mesh=mesh,
        out_type=jax.ShapeDtypeStruct((bs, table.shape[1]), jnp.float32),
        scratch_types=[
            pltpu.VMEM((bs,), jnp.int32),
            pltpu.VMEM((bs, table.shape[1]), jnp.float32),
            pltpu.SemaphoreType.DMA,
        ],
    )(_gather_body)
    return k(table, idx)


def _copy_body(src_ref, mask_ref, xo_ref, mo_ref):
    b = pl.program_id(0)
    seq = src_ref.shape[1]
    xo_ref[:, _PMT:, :] = src_ref[:, :seq, :]

    @pl.when(b == 0)
    def _():
        nb = mask_ref.shape[0]
        mo_ref[:, :_PMT] = jnp.zeros((nb, _PMT), mask_ref.dtype)
        mo_ref[:, _PMT:] = mask_ref[...]


def _tc_copy(src_embs, src_mask):
    bs, seq, d = src_embs.shape
    out_seq = seq + _PMT
    return pl.pallas_call(
        _copy_body,
        grid=(bs,),
        in_specs=[
            pl.BlockSpec((1, seq, d), lambda b: (b, 0, 0)),
            pl.BlockSpec((bs, seq), lambda b: (0, 0)),
        ],
        out_specs=[
            pl.BlockSpec((1, out_seq, d), lambda b: (b, 0, 0)),
            pl.BlockSpec((bs, out_seq), lambda b: (0, 0)),
        ],
        out_shape=[
            jax.ShapeDtypeStruct((bs, out_seq, d), src_embs.dtype),
            jax.ShapeDtypeStruct((bs, out_seq), src_mask.dtype),
        ],
    )(src_embs, src_mask)


def _patch_body(p_ref, xin_ref, xo_ref):
    for r in range(_PMT):
        xo_ref[:, r, :] = p_ref[:, pl.ds(r * _D, _D)]


def _tc_patch(p, x):
    bs, out_seq, d = x.shape
    return pl.pallas_call(
        _patch_body,
        grid=(1,),
        in_specs=[
            pl.BlockSpec((bs, _PMT * d), lambda n: (0, 0)),
            pl.BlockSpec(memory_space=pl.ANY),
        ],
        out_specs=pl.BlockSpec((bs, _PMT, d), lambda n: (0, 0, 0)),
        out_shape=jax.ShapeDtypeStruct(x.shape, x.dtype),
        input_output_aliases={1: 0},
    )(p, x)


def kernel(src_embs, src_mask, tissue_idx, prompt_emb):
    p = _sc_gather(prompt_emb, tissue_idx)
    x_partial, new_mask = _tc_copy(src_embs, src_mask)
    x = _tc_patch(p, x_partial)
    return (x, new_mask)


# speedup vs baseline: 3.0781x; 1.0217x over previous
import functools

import jax
import jax.numpy as jnp
from jax import lax
from jax.experimental import pallas as pl
from jax.experimental.pallas import tpu as pltpu
from jax.experimental.pallas import tpu_sc as plsc

_PMT = 16
_D = 1024
_S = 688
_SB = _S // _PMT


def _gather_body(table_hbm, idx_hbm, out_hbm, idx_s, sem):
    core = lax.axis_index("c")
    nb = out_hbm.shape[0]

    @pl.when(core == 0)
    def _():
        pltpu.sync_copy(idx_hbm, idx_s)
        for b in range(nb):
            t = idx_s[b]
            pltpu.sync_copy(table_hbm.at[pl.ds(t, 1)],
                            out_hbm.at[pl.ds(b, 1)])


def _sc_gather(table, idx):
    bs = idx.shape[0]
    mesh = plsc.ScalarSubcoreMesh(axis_name="c", num_cores=2)
    k = functools.partial(
        pl.kernel,
        mesh=mesh,
        out_type=jax.ShapeDtypeStruct((bs, table.shape[1]), jnp.float32),
        scratch_types=[
            pltpu.SMEM((bs,), jnp.int32),
            pltpu.SemaphoreType.DMA,
        ],
    )(_gather_body)
    return k(table, idx)


def _copy_body(src_ref, mask_ref, xo_ref, mo_ref):
    b = pl.program_id(0)
    seq = src_ref.shape[1]
    xo_ref[:, _PMT:, :] = src_ref[:, :seq, :]

    @pl.when(b == 0)
    def _():
        nb = mask_ref.shape[0]
        mo_ref[:, :_PMT] = jnp.zeros((nb, _PMT), mask_ref.dtype)
        mo_ref[:, _PMT:] = mask_ref[...]


def _tc_copy(src_embs, src_mask):
    bs, seq, d = src_embs.shape
    out_seq = seq + _PMT
    return pl.pallas_call(
        _copy_body,
        grid=(bs,),
        in_specs=[
            pl.BlockSpec((1, seq, d), lambda b: (b, 0, 0)),
            pl.BlockSpec((bs, seq), lambda b: (0, 0)),
        ],
        out_specs=[
            pl.BlockSpec((1, out_seq, d), lambda b: (b, 0, 0)),
            pl.BlockSpec((bs, out_seq), lambda b: (0, 0)),
        ],
        out_shape=[
            jax.ShapeDtypeStruct((bs, out_seq, d), src_embs.dtype),
            jax.ShapeDtypeStruct((bs, out_seq), src_mask.dtype),
        ],
    )(src_embs, src_mask)


def _patch_body(p_ref, xin_ref, xo_ref):
    for r in range(_PMT):
        xo_ref[:, r, :] = p_ref[:, pl.ds(r * _D, _D)]


def _tc_patch(p, x):
    bs, out_seq, d = x.shape
    return pl.pallas_call(
        _patch_body,
        grid=(1,),
        in_specs=[
            pl.BlockSpec((bs, _PMT * d), lambda n: (0, 0)),
            pl.BlockSpec(memory_space=pl.ANY),
        ],
        out_specs=pl.BlockSpec((bs, _PMT, d), lambda n: (0, 0, 0)),
        out_shape=jax.ShapeDtypeStruct(x.shape, x.dtype),
        input_output_aliases={1: 0},
    )(p, x)


def kernel(src_embs, src_mask, tissue_idx, prompt_emb):
    p = _sc_gather(prompt_emb, tissue_idx)
    x_partial, new_mask = _tc_copy(src_embs, src_mask)
    x = _tc_patch(p, x_partial)
    return (x, new_mask)


# speedup vs baseline: 3.0822x; 1.0013x over previous
import functools

import jax
import jax.numpy as jnp
from jax import lax
from jax.experimental import pallas as pl
from jax.experimental.pallas import tpu as pltpu
from jax.experimental.pallas import tpu_sc as plsc

_PMT = 16
_D = 1024
_S = 688
_SB = _S // _PMT


def _gather_body(table_hbm, idx_hbm, out_hbm, idx_s, sem):
    core = lax.axis_index("c")
    nb = out_hbm.shape[0]
    half = nb // 2

    pltpu.sync_copy(idx_hbm, idx_s)
    copies = []
    for j in range(half):
        b = core * half + j
        t = idx_s[b]
        copies.append(
            pltpu.async_copy(table_hbm.at[pl.ds(t, 1)],
                             out_hbm.at[pl.ds(b, 1)], sem))
    for c in copies:
        c.wait()


def _sc_gather(table, idx):
    bs = idx.shape[0]
    mesh = plsc.ScalarSubcoreMesh(axis_name="c", num_cores=2)
    k = functools.partial(
        pl.kernel,
        mesh=mesh,
        out_type=jax.ShapeDtypeStruct((bs, table.shape[1]), jnp.float32),
        scratch_types=[
            pltpu.SMEM((bs,), jnp.int32),
            pltpu.SemaphoreType.DMA,
        ],
    )(_gather_body)
    return k(table, idx)


def _copy_body(src_ref, mask_ref, xo_ref, mo_ref):
    b = pl.program_id(0)
    seq = src_ref.shape[1]
    xo_ref[:, _PMT:, :] = src_ref[:, :seq, :]

    @pl.when(b == 0)
    def _():
        nb = mask_ref.shape[0]
        mo_ref[:, :_PMT] = jnp.zeros((nb, _PMT), mask_ref.dtype)
        mo_ref[:, _PMT:] = mask_ref[...]


def _tc_copy(src_embs, src_mask):
    bs, seq, d = src_embs.shape
    out_seq = seq + _PMT
    return pl.pallas_call(
        _copy_body,
        grid=(bs,),
        in_specs=[
            pl.BlockSpec((1, seq, d), lambda b: (b, 0, 0)),
            pl.BlockSpec((bs, seq), lambda b: (0, 0)),
        ],
        out_specs=[
            pl.BlockSpec((1, out_seq, d), lambda b: (b, 0, 0)),
            pl.BlockSpec((bs, out_seq), lambda b: (0, 0)),
        ],
        out_shape=[
            jax.ShapeDtypeStruct((bs, out_seq, d), src_embs.dtype),
            jax.ShapeDtypeStruct((bs, out_seq), src_mask.dtype),
        ],
    )(src_embs, src_mask)


def _patch_body(p_ref, xin_ref, xo_ref):
    for r in range(_PMT):
        xo_ref[:, r, :] = p_ref[:, pl.ds(r * _D, _D)]


def _tc_patch(p, x):
    bs, out_seq, d = x.shape
    return pl.pallas_call(
        _patch_body,
        grid=(1,),
        in_specs=[
            pl.BlockSpec((bs, _PMT * d), lambda n: (0, 0)),
            pl.BlockSpec(memory_space=pl.ANY),
        ],
        out_specs=pl.BlockSpec((bs, _PMT, d), lambda n: (0, 0, 0)),
        out_shape=jax.ShapeDtypeStruct(x.shape, x.dtype),
        input_output_aliases={1: 0},
    )(p, x)


def kernel(src_embs, src_mask, tissue_idx, prompt_emb):
    p = _sc_gather(prompt_emb, tissue_idx)
    x_partial, new_mask = _tc_copy(src_embs, src_mask)
    x = _tc_patch(p, x_partial)
    return (x, new_mask)


# speedup vs baseline: 3.0835x; 1.0004x over previous
import functools

import jax
import jax.numpy as jnp
from jax import lax
from jax.experimental import pallas as pl
from jax.experimental.pallas import tpu as pltpu
from jax.experimental.pallas import tpu_sc as plsc

_PMT = 16
_D = 1024


def _gather_body(table_hbm, idx_hbm, out_hbm, idx_s, sem):
    core = lax.axis_index("c")
    nb = out_hbm.shape[0]
    half = nb // 2

    pltpu.sync_copy(idx_hbm, idx_s)
    copies = []
    for j in range(half):
        b = core * half + j
        t = idx_s[b]
        copies.append(
            pltpu.async_copy(table_hbm.at[pl.ds(t, 1)],
                             out_hbm.at[pl.ds(b, 1)], sem))
    for c in copies:
        c.wait()


def _sc_gather(table, idx):
    bs = idx.shape[0]
    mesh = plsc.ScalarSubcoreMesh(axis_name="c", num_cores=2)
    k = functools.partial(
        pl.kernel,
        mesh=mesh,
        out_type=jax.ShapeDtypeStruct((bs, table.shape[1]), jnp.float32),
        scratch_types=[
            pltpu.SMEM((bs,), jnp.int32),
            pltpu.SemaphoreType.DMA,
        ],
    )(_gather_body)
    return k(table, idx)


def _copy_body(src_ref, mask_ref, xo_ref, mo_ref):
    b = pl.program_id(0)
    seq = src_ref.shape[1]
    xo_ref[:, _PMT:, :] = src_ref[:, :seq, :]

    @pl.when(b == 0)
    def _():
        nb = mask_ref.shape[0]
        mo_ref[:, :_PMT] = jnp.zeros((nb, _PMT), mask_ref.dtype)
        mo_ref[:, _PMT:] = mask_ref[...]


def _tc_copy(src_embs, src_mask):
    bs, seq, d = src_embs.shape
    out_seq = seq + _PMT
    return pl.pallas_call(
        _copy_body,
        grid=(bs,),
        in_specs=[
            pl.BlockSpec((1, seq, d), lambda b: (b, 0, 0)),
            pl.BlockSpec((bs, seq), lambda b: (0, 0)),
        ],
        out_specs=[
            pl.BlockSpec((1, out_seq, d), lambda b: (b, 0, 0)),
            pl.BlockSpec((bs, out_seq), lambda b: (0, 0)),
        ],
        out_shape=[
            jax.ShapeDtypeStruct((bs, out_seq, d), src_embs.dtype),
            jax.ShapeDtypeStruct((bs, out_seq), src_mask.dtype),
        ],
    )(src_embs, src_mask)


def _patch_body(p_ref, xin_ref, xo_ref):
    for r in range(_PMT):
        xo_ref[:, r, :] = p_ref[:, pl.ds(r * _D, _D)]


def _tc_patch(p, x):
    bs, out_seq, d = x.shape
    return pl.pallas_call(
        _patch_body,
        grid=(1,),
        in_specs=[
            pl.BlockSpec((bs, _PMT * d), lambda n: (0, 0)),
            pl.BlockSpec(memory_space=pl.ANY),
        ],
        out_specs=pl.BlockSpec((bs, _PMT, d), lambda n: (0, 0, 0)),
        out_shape=jax.ShapeDtypeStruct(x.shape, x.dtype),
        input_output_aliases={1: 0},
    )(p, x)


def kernel(src_embs, src_mask, tissue_idx, prompt_emb):
    p = _sc_gather(prompt_emb, tissue_idx)
    x_partial, new_mask = _tc_copy(src_embs, src_mask)
    x = _tc_patch(p, x_partial)
    return (x, new_mask)
